# baseline (speedup 1.0000x reference)
import functools

import jax
import jax.numpy as jnp
from jax import lax
from jax.experimental import pallas as pl
from jax.experimental.pallas import tpu as pltpu
from jax.experimental.pallas import tpu_sc as plsc

B, N, K = 32, 512, 16
D = 256
BN = B * N
LANES = 16
DPACK = D // 2
STT = DPACK + 1
NH = N // 2


SLAB = 8
NSLAB = N // SLAB
NG = 4


def _knn_phase(d_hbm, b, d_v0, d_v1, idx_v, sem):
    bufs = [d_v0, d_v1]

    def start(s):
        return pltpu.async_copy(
            d_hbm.at[b, pl.ds(s * SLAB * N, SLAB * N)], bufs[s % 2], sem)

    cp = start(0)
    for s in range(NSLAB):
        cp.wait()
        if s + 1 < NSLAB:
            cp = start(s + 1)
        d_v = bufs[s % 2]

        @plsc.parallel_loop(0, SLAB // NG)
        def ngroup(gg):
            i0 = gg * NG
            iota = lax.iota(jnp.int32, LANES)
            tops = [plsc.sort_key_val(d_v[pl.ds((i0 + t) * N, LANES)], iota)
                    for t in range(NG)]

            def chunk_body(c, carry):
                out = []
                for t, (tk, tv) in enumerate(carry):
                    ck, cv = plsc.sort_key_val(
                        d_v[pl.ds((i0 + t) * N + c * LANES, LANES)],
                        iota + c * LANES)
                    rk = jnp.flip(ck, 0)
                    rv = jnp.flip(cv, 0)
                    m = tk <= rk
                    out.append(plsc.sort_key_val(jnp.where(m, tk, rk),
                                                 jnp.where(m, tv, rv)))
                return tuple(out)

            tops = lax.fori_loop(1, N // LANES, chunk_body, tuple(tops))
            for t, (tk, tv) in enumerate(tops):
                gi = s * SLAB + i0 + t
                plsc.store_scatter(idx_v, [iota * N + gi], tv)


def _pack_words(a):
    abf = a.astype(jnp.bfloat16)
    lo = lax.bitcast_convert_type(abf[:, :DPACK], jnp.uint16).astype(jnp.uint32)
    hi = lax.bitcast_convert_type(abf[:, DPACK:], jnp.uint16).astype(jnp.uint32)
    w = lax.bitcast_convert_type(lo | (hi << 16), jnp.int32)
    return jnp.concatenate([w, jnp.zeros((N, STT - DPACK), jnp.int32)], axis=1)


def _unpack_words(g):
    w = lax.bitcast_convert_type(g[:, :DPACK], jnp.uint32)
    lo = lax.bitcast_convert_type((w & 0xFFFF).astype(jnp.uint16), jnp.bfloat16)
    hi = lax.bitcast_convert_type(
        lax.shift_right_logical(w, jnp.uint32(16)).astype(jnp.uint16),
        jnp.bfloat16)
    return jnp.concatenate([lo, hi], axis=1).astype(jnp.float32)


def _mm_first_body(x_ref, wa_ref, wd_ref, b_ref,
                   pxc, pyc, pzc, pxr, pyr, pzr, a_ref, c_ref, d_ref):
    x = x_ref[...]
    a = jnp.dot(x, wa_ref[...], preferred_element_type=jnp.float32)
    c = jnp.dot(x, wd_ref[...], preferred_element_type=jnp.float32) + b_ref[...]
    a_ref[...] = _pack_words(a)
    c_ref[...] = c
    dx = pxc[0] - pxr[0]
    dy = pyc[0] - pyr[0]
    dz = pzc[0] - pzr[0]
    dd = dx * dx + dy * dy + dz * dz
    iota_i = lax.broadcasted_iota(jnp.int32, (N, N), 0)
    iota_j = lax.broadcasted_iota(jnp.int32, (N, N), 1)
    d_ref[0] = jnp.where(iota_j == iota_i, 1e10, dd)


def _mm_first(x, wa, wd, b, pos):
    row = pl.BlockSpec((N, D), lambda i: (i, 0))
    arow = pl.BlockSpec((N, STT), lambda i: (i, 0))
    wspec = pl.BlockSpec((D, D), lambda i: (0, 0))
    bspec = pl.BlockSpec((1, D), lambda i: (0, 0))
    cols = [pos[:, :, i][:, :, None] for i in range(3)]
    rows = [pos[:, :, i][:, None, :] for i in range(3)]
    spec_c = pl.BlockSpec((1, N, 1), lambda i: (i, 0, 0))
    spec_r = pl.BlockSpec((1, 1, N), lambda i: (i, 0, 0))
    return pl.pallas_call(
        _mm_first_body,
        grid=(B,),
        in_specs=[row, wspec, wspec, bspec] + [spec_c] * 3 + [spec_r] * 3,
        out_specs=[arow, row, pl.BlockSpec((1, N, N), lambda i: (i, 0, 0))],
        out_shape=[jax.ShapeDtypeStruct((BN, STT), jnp.int32),
                   jax.ShapeDtypeStruct((BN, D), jnp.float32),
                   jax.ShapeDtypeStruct((B, N, N), jnp.float32)],
    )(x, wa, wd, b[None, :], *cols, *rows)


def _mm_mid_body(g_ref, cp_ref, wa_ref, wd_ref, b_ref, x_ref, a_ref, c_ref):
    x = jnp.maximum(_unpack_words(g_ref[...]) + cp_ref[...], 0.0)
    x_ref[...] = x
    a = jnp.dot(x, wa_ref[...], preferred_element_type=jnp.float32)
    c = jnp.dot(x, wd_ref[...], preferred_element_type=jnp.float32) + b_ref[...]
    a_ref[...] = _pack_words(a)
    c_ref[...] = c


def _mm_mid(g, c_prev, wa, wd, b):
    row = pl.BlockSpec((N, D), lambda i: (i, 0))
    arow = pl.BlockSpec((N, STT), lambda i: (i, 0))
    wspec = pl.BlockSpec((D, D), lambda i: (0, 0))
    bspec = pl.BlockSpec((1, D), lambda i: (0, 0))
    return pl.pallas_call(
        _mm_mid_body,
        grid=(B,),
        in_specs=[arow, row, wspec, wspec, bspec],
        out_specs=[row, arow, row],
        out_shape=[jax.ShapeDtypeStruct((BN, D), jnp.float32),
                   jax.ShapeDtypeStruct((BN, STT), jnp.int32),
                   jax.ShapeDtypeStruct((BN, D), jnp.float32)],
    )(g, c_prev, wa, wd, b[None, :])


def _final_body(x0_ref, x1_ref, x2_ref, g_ref, c_ref, out_ref):
    x3 = jnp.maximum(_unpack_words(g_ref[...]) + c_ref[...], 0.0)
    out_ref[:, 0 * D:1 * D] = x0_ref[...]
    out_ref[:, 1 * D:2 * D] = x1_ref[...]
    out_ref[:, 2 * D:3 * D] = x2_ref[...]
    out_ref[:, 3 * D:4 * D] = x3


def _final(x0, x1, x2, g, c3):
    row = pl.BlockSpec((N, D), lambda i: (i, 0))
    arow = pl.BlockSpec((N, STT), lambda i: (i, 0))
    return pl.pallas_call(
        _final_body,
        grid=(B,),
        in_specs=[row, row, row, arow, row],
        out_specs=pl.BlockSpec((N, 4 * D), lambda i: (i, 0)),
        out_shape=jax.ShapeDtypeStruct((BN, 4 * D), jnp.float32),
    )(x0, x1, x2, g, c3)


def _gmax_phase(a_hbm, o_hbm, b, idx_v, tab_v, out_v, sem):
    pltpu.sync_copy(a_hbm.at[b], tab_v)
    for h in range(2):

        @plsc.parallel_loop(0, NH // LANES)
        def gbody(g):
            i0 = h * NH + g * LANES
            ovec = (g * LANES + lax.iota(jnp.int32, LANES)) * STT
            rows = [idx_v[pl.ds(k * N + i0, LANES)] * STT for k in range(K)]

            @plsc.parallel_loop(0, DPACK, unroll=4)
            def dbody(w):
                acc = plsc.bitcast(
                    plsc.load_gather(tab_v, [rows[0] + w]), jnp.bfloat16)
                for k in range(1, K):
                    v = plsc.bitcast(
                        plsc.load_gather(tab_v, [rows[k] + w]), jnp.bfloat16)
                    acc = jnp.maximum(acc, v)
                plsc.store_scatter(out_v, [ovec + w],
                                   plsc.bitcast(acc, jnp.int32))

        pltpu.sync_copy(out_v, o_hbm.at[b, h])


def _worker_id():
    return lax.axis_index("s") * 2 + lax.axis_index("c")


def _gmax_body(a_hbm, idx_hbm, o_hbm, idx_v, tab_v, out_v, sem):
    b = _worker_id()
    pltpu.sync_copy(idx_hbm.at[b], idx_v)
    _gmax_phase(a_hbm, o_hbm, b, idx_v, tab_v, out_v, sem)


def _knn_gmax_body(d_hbm, a_hbm, idx_hbm, o_hbm,
                   d_v0, d_v1, idx_v, tab_v, out_v, sem, sem2):
    b = _worker_id()
    _knn_phase(d_hbm, b, d_v0, d_v1, idx_v, sem)
    cp = pltpu.async_copy(idx_v, idx_hbm.at[b], sem2)
    _gmax_phase(a_hbm, o_hbm, b, idx_v, tab_v, out_v, sem)
    cp.wait()


_SC_MESH = plsc.VectorSubcoreMesh(core_axis_name="c", subcore_axis_name="s")
_GM_SCRATCH = [
    pltpu.VMEM((K * N,), jnp.int32),
    pltpu.VMEM((N * STT,), jnp.int32),
    pltpu.VMEM((NH * STT,), jnp.int32),
    pltpu.SemaphoreType.DMA,
]


def _gmax(aw, idxT):
    f = functools.partial(
        pl.kernel,
        out_type=jax.ShapeDtypeStruct((B, 2, NH * STT), jnp.int32),
        mesh=_SC_MESH,
        compiler_params=pltpu.CompilerParams(needs_layout_passes=False),
        scratch_types=_GM_SCRATCH,
    )(_gmax_body)
    o = f(aw.reshape(B, N * STT), idxT)
    return o.reshape(BN, STT)


def _knn_gmax(d, aw):
    f = functools.partial(
        pl.kernel,
        out_type=[jax.ShapeDtypeStruct((B, K * N), jnp.int32),
                  jax.ShapeDtypeStruct((B, 2, NH * STT), jnp.int32)],
        mesh=_SC_MESH,
        compiler_params=pltpu.CompilerParams(needs_layout_passes=False),
        scratch_types=[
            pltpu.VMEM((SLAB * N,), jnp.float32),
            pltpu.VMEM((SLAB * N,), jnp.float32),
        ] + _GM_SCRATCH + [pltpu.SemaphoreType.DMA],
    )(_knn_gmax_body)
    idxT, o = f(d, aw.reshape(B, N * STT))
    return idxT, o.reshape(BN, STT)


def kernel(rois, pooled_features, W1, b1, W2, b2, W3, b3):
    pos = rois[:, :, :3]
    x0 = pooled_features.reshape(BN, D)

    was = [W[:D] for W in (W1, W2, W3)]
    wds = [W[D:] - W[:D] for W in (W1, W2, W3)]
    bs = [b1, b2, b3]

    a, c, d = _mm_first(x0, was[0], wds[0], bs[0], pos)
    idxT, g = _knn_gmax(d.reshape(B, N * N), a)
    x1, a, c = _mm_mid(g, c, was[1], wds[1], bs[1])
    g = _gmax(a, idxT)
    x2, a, c = _mm_mid(g, c, was[2], wds[2], bs[2])
    g = _gmax(a, idxT)
    return _final(x0, x1, x2, g, c)


# speedup vs baseline: 1.0941x; 1.0941x over previous
import functools

import jax
import jax.numpy as jnp
from jax import lax
from jax.experimental import pallas as pl
from jax.experimental.pallas import tpu as pltpu
from jax.experimental.pallas import tpu_sc as plsc

B, N, K = 32, 512, 16
D = 256
BN = B * N
LANES = 16
DPACK = D // 2
STT = DPACK + 1
NH = N // 2


def _dist_body(pxc, pyc, pzc, pxr, pyr, pzr, d_ref):
    dx = pxc[0] - pxr[0]
    dy = pyc[0] - pyr[0]
    dz = pzc[0] - pzr[0]
    dd = dx * dx + dy * dy + dz * dz
    iota_i = lax.broadcasted_iota(jnp.int32, (N, N), 0)
    iota_j = lax.broadcasted_iota(jnp.int32, (N, N), 1)
    d_ref[0] = jnp.where(iota_j == iota_i, 1e10, dd)


def _dist(pos):
    cols = [pos[:, :, i][:, :, None] for i in range(3)]
    rows = [pos[:, :, i][:, None, :] for i in range(3)]
    spec_c = pl.BlockSpec((1, N, 1), lambda b: (b, 0, 0))
    spec_r = pl.BlockSpec((1, 1, N), lambda b: (b, 0, 0))
    return pl.pallas_call(
        _dist_body,
        grid=(B,),
        in_specs=[spec_c] * 3 + [spec_r] * 3,
        out_specs=pl.BlockSpec((1, N, N), lambda b: (b, 0, 0)),
        out_shape=jax.ShapeDtypeStruct((B, N, N), jnp.float32),
    )(cols[0], cols[1], cols[2], rows[0], rows[1], rows[2])


SLAB = 64
NSLAB = N // SLAB
NG = 4


def _knn_sc_body(d_hbm, idx_hbm, d_v0, d_v1, idx_v, sem):
    cid = lax.axis_index("c")
    sid = lax.axis_index("s")
    b = sid * 2 + cid
    bufs = [d_v0, d_v1]

    def start(s):
        return pltpu.async_copy(
            d_hbm.at[b, pl.ds(s * SLAB * N, SLAB * N)], bufs[s % 2], sem)

    cp = start(0)
    for s in range(NSLAB):
        cp.wait()
        if s + 1 < NSLAB:
            cp = start(s + 1)
        d_v = bufs[s % 2]

        @plsc.parallel_loop(0, SLAB // NG)
        def ngroup(gg):
            i0 = gg * NG
            iota = lax.iota(jnp.int32, LANES)
            tops = [plsc.sort_key_val(d_v[pl.ds((i0 + t) * N, LANES)], iota)
                    for t in range(NG)]

            def chunk_body(c, carry):
                out = []
                for t, (tk, tv) in enumerate(carry):
                    ck, cv = plsc.sort_key_val(
                        d_v[pl.ds((i0 + t) * N + c * LANES, LANES)],
                        iota + c * LANES)
                    rk = jnp.flip(ck, 0)
                    rv = jnp.flip(cv, 0)
                    m = tk <= rk
                    out.append(plsc.sort_key_val(jnp.where(m, tk, rk),
                                                 jnp.where(m, tv, rv)))
                return tuple(out)

            tops = lax.fori_loop(1, N // LANES, chunk_body, tuple(tops))
            for t, (tk, tv) in enumerate(tops):
                gi = s * SLAB + i0 + t
                plsc.store_scatter(idx_v, [iota * N + gi], tv)

    pltpu.sync_copy(idx_v, idx_hbm.at[b])


def _knn(pos):
    d = _dist(pos).reshape(B, N * N)
    mesh = plsc.VectorSubcoreMesh(core_axis_name="c", subcore_axis_name="s")
    f = functools.partial(
        pl.kernel,
        out_type=jax.ShapeDtypeStruct((B, K * N), jnp.int32),
        mesh=mesh,
        compiler_params=pltpu.CompilerParams(needs_layout_passes=False),
        scratch_types=[
            pltpu.VMEM((SLAB * N,), jnp.float32),
            pltpu.VMEM((SLAB * N,), jnp.float32),
            pltpu.VMEM((K * N,), jnp.int32),
            pltpu.SemaphoreType.DMA,
        ],
    )(_knn_sc_body)
    return f(d)


def _pack_words(a):
    abf = a.astype(jnp.bfloat16)
    lo = lax.bitcast_convert_type(abf[:, :DPACK], jnp.uint16).astype(jnp.uint32)
    hi = lax.bitcast_convert_type(abf[:, DPACK:], jnp.uint16).astype(jnp.uint32)
    w = lax.bitcast_convert_type(lo | (hi << 16), jnp.int32)
    return jnp.concatenate([w, jnp.zeros((N, STT - DPACK), jnp.int32)], axis=1)


def _unpack_words(g):
    w = lax.bitcast_convert_type(g[:, :DPACK], jnp.uint32)
    lo = lax.bitcast_convert_type((w & 0xFFFF).astype(jnp.uint16), jnp.bfloat16)
    hi = lax.bitcast_convert_type(
        lax.shift_right_logical(w, jnp.uint32(16)).astype(jnp.uint16),
        jnp.bfloat16)
    return jnp.concatenate([lo, hi], axis=1).astype(jnp.float32)


def _mm_first_body(x_ref, wa_ref, wd_ref, b_ref, a_ref, c_ref):
    x = x_ref[...]
    a = jnp.dot(x, wa_ref[...], preferred_element_type=jnp.float32)
    c = jnp.dot(x, wd_ref[...], preferred_element_type=jnp.float32) + b_ref[...]
    a_ref[...] = _pack_words(a)
    c_ref[...] = c


def _mm_first(x, wa, wd, b):
    row = pl.BlockSpec((N, D), lambda i: (i, 0))
    arow = pl.BlockSpec((N, STT), lambda i: (i, 0))
    wspec = pl.BlockSpec((D, D), lambda i: (0, 0))
    bspec = pl.BlockSpec((1, D), lambda i: (0, 0))
    return pl.pallas_call(
        _mm_first_body,
        grid=(B,),
        in_specs=[row, wspec, wspec, bspec],
        out_specs=[arow, row],
        out_shape=[jax.ShapeDtypeStruct((BN, STT), jnp.int32),
                   jax.ShapeDtypeStruct((BN, D), jnp.float32)],
    )(x, wa, wd, b[None, :])


def _mm_mid_body(g_ref, cp_ref, wa_ref, wd_ref, b_ref, x_ref, a_ref, c_ref):
    x = jnp.maximum(_unpack_words(g_ref[...]) + cp_ref[...], 0.0)
    x_ref[...] = x
    a = jnp.dot(x, wa_ref[...], preferred_element_type=jnp.float32)
    c = jnp.dot(x, wd_ref[...], preferred_element_type=jnp.float32) + b_ref[...]
    a_ref[...] = _pack_words(a)
    c_ref[...] = c


def _mm_mid(g, c_prev, wa, wd, b):
    row = pl.BlockSpec((N, D), lambda i: (i, 0))
    arow = pl.BlockSpec((N, STT), lambda i: (i, 0))
    wspec = pl.BlockSpec((D, D), lambda i: (0, 0))
    bspec = pl.BlockSpec((1, D), lambda i: (0, 0))
    return pl.pallas_call(
        _mm_mid_body,
        grid=(B,),
        in_specs=[arow, row, wspec, wspec, bspec],
        out_specs=[row, arow, row],
        out_shape=[jax.ShapeDtypeStruct((BN, D), jnp.float32),
                   jax.ShapeDtypeStruct((BN, STT), jnp.int32),
                   jax.ShapeDtypeStruct((BN, D), jnp.float32)],
    )(g, c_prev, wa, wd, b[None, :])


def _final_body(x0_ref, x1_ref, x2_ref, g_ref, c_ref, out_ref):
    x3 = jnp.maximum(_unpack_words(g_ref[...]) + c_ref[...], 0.0)
    out_ref[:, 0 * D:1 * D] = x0_ref[...]
    out_ref[:, 1 * D:2 * D] = x1_ref[...]
    out_ref[:, 2 * D:3 * D] = x2_ref[...]
    out_ref[:, 3 * D:4 * D] = x3


def _final(x0, x1, x2, g, c3):
    row = pl.BlockSpec((N, D), lambda i: (i, 0))
    arow = pl.BlockSpec((N, STT), lambda i: (i, 0))
    return pl.pallas_call(
        _final_body,
        grid=(B,),
        in_specs=[row, row, row, arow, row],
        out_specs=pl.BlockSpec((N, 4 * D), lambda i: (i, 0)),
        out_shape=jax.ShapeDtypeStruct((BN, 4 * D), jnp.float32),
    )(x0, x1, x2, g, c3)


def _gmax_body(a_hbm, idx_hbm, o_hbm, idx_v, tab_v, out_v, sem):
    cid = lax.axis_index("c")
    sid = lax.axis_index("s")
    b = sid * 2 + cid
    pltpu.sync_copy(idx_hbm.at[b], idx_v)
    pltpu.sync_copy(a_hbm.at[b], tab_v)
    for h in range(2):

        @plsc.parallel_loop(0, NH // LANES)
        def gbody(g):
            i0 = h * NH + g * LANES
            ovec = (g * LANES + lax.iota(jnp.int32, LANES)) * STT
            rows = [idx_v[pl.ds(k * N + i0, LANES)] * STT for k in range(K)]

            @plsc.parallel_loop(0, DPACK, unroll=4)
            def dbody(w):
                acc = plsc.bitcast(
                    plsc.load_gather(tab_v, [rows[0] + w]), jnp.bfloat16)
                for k in range(1, K):
                    v = plsc.bitcast(
                        plsc.load_gather(tab_v, [rows[k] + w]), jnp.bfloat16)
                    acc = jnp.maximum(acc, v)
                plsc.store_scatter(out_v, [ovec + w],
                                   plsc.bitcast(acc, jnp.int32))

        pltpu.sync_copy(out_v, o_hbm.at[b, h])


def _gmax(aw, idxT):
    mesh = plsc.VectorSubcoreMesh(core_axis_name="c", subcore_axis_name="s")
    f = functools.partial(
        pl.kernel,
        out_type=jax.ShapeDtypeStruct((B, 2, NH * STT), jnp.int32),
        mesh=mesh,
        compiler_params=pltpu.CompilerParams(needs_layout_passes=False),
        scratch_types=[
            pltpu.VMEM((K * N,), jnp.int32),
            pltpu.VMEM((N * STT,), jnp.int32),
            pltpu.VMEM((NH * STT,), jnp.int32),
            pltpu.SemaphoreType.DMA,
        ],
    )(_gmax_body)
    o = f(aw.reshape(B, N * STT), idxT)
    return o.reshape(BN, STT)


def kernel(rois, pooled_features, W1, b1, W2, b2, W3, b3):
    pos = rois[:, :, :3]
    x0 = pooled_features.reshape(BN, D)

    idxT = _knn(pos)

    was = [W[:D] for W in (W1, W2, W3)]
    wds = [W[D:] - W[:D] for W in (W1, W2, W3)]
    bs = [b1, b2, b3]

    a, c = _mm_first(x0, was[0], wds[0], bs[0])
    g = _gmax(a, idxT)
    x1, a, c = _mm_mid(g, c, was[1], wds[1], bs[1])
    g = _gmax(a, idxT)
    x2, a, c = _mm_mid(g, c, was[2], wds[2], bs[2])
    g = _gmax(a, idxT)
    return _final(x0, x1, x2, g, c)
